# Initial kernel scaffold; baseline (speedup 1.0000x reference)
import jax
import jax.numpy as jnp
from jax.experimental import pallas as pl


def kernel(features, edge_index, W1, attn_l1, attn_r1, b1, W2, attn_l2, attn_r2, b2):
    raise NotImplementedError("write your pallas kernel here")


# speedup vs baseline: 1.2560x; 1.2560x over previous
import functools

import jax
import jax.numpy as jnp
from jax.experimental import pallas as pl
from jax.experimental.pallas import tpu as pltpu

N = 10000
DIM = 256
BN = 1000


def _mm_body(xa_ref, xb_ref, wa_ref, wb_ref, a_ref, fa_ref, fb_ref, s_ref):
    feat = jnp.dot(xa_ref[...], wa_ref[...], preferred_element_type=jnp.float32)
    feat += jnp.dot(xb_ref[...], wb_ref[...], preferred_element_type=jnp.float32)
    fa_ref[...] = feat[:, :128]
    fb_ref[...] = feat[:, 128:]
    s_ref[...] = jnp.dot(feat, a_ref[...], preferred_element_type=jnp.float32)


@jax.jit
def _matmul_scores(xa, xb, wa, wb, a):
    n = xa.shape[0]
    grid = (n // BN,)
    return pl.pallas_call(
        _mm_body,
        grid=grid,
        in_specs=[
            pl.BlockSpec((BN, 128), lambda i: (i, 0)),
            pl.BlockSpec((BN, 128), lambda i: (i, 0)),
            pl.BlockSpec((128, DIM), lambda i: (0, 0)),
            pl.BlockSpec((128, DIM), lambda i: (0, 0)),
            pl.BlockSpec((DIM, 128), lambda i: (0, 0)),
        ],
        out_specs=[
            pl.BlockSpec((BN, 128), lambda i: (i, 0)),
            pl.BlockSpec((BN, 128), lambda i: (i, 0)),
            pl.BlockSpec((BN, 128), lambda i: (i, 0)),
        ],
        out_shape=[
            jax.ShapeDtypeStruct((n, 128), jnp.float32),
            jax.ShapeDtypeStruct((n, 128), jnp.float32),
            jax.ShapeDtypeStruct((n, 128), jnp.float32),
        ],
    )(xa, xb, wa, wb, a)


def _edge_phase(feat_a, feat_b, el, er, src, dst, bias, apply_elu):
    feat = jnp.concatenate([feat_a, feat_b], axis=1)
    e = el[src] + er[dst]
    e = jnp.where(e > 0, e, 0.2 * e)
    ee = jnp.exp(e)
    denom = jax.ops.segment_sum(ee, dst, num_segments=N)
    alpha = ee / jnp.maximum(denom[dst], 1e-9)
    out = jax.ops.segment_sum(feat[src] * alpha[:, None], dst, num_segments=N)
    out = out + bias
    if apply_elu:
        out = jnp.where(out > 0, out, jnp.exp(jnp.minimum(out, 0.0)) - 1.0)
    return out[:, :128], out[:, 128:]


def _pack_attn(attn_l, attn_r):
    a = jnp.zeros((DIM, 128), jnp.float32)
    a = a.at[:, 0].set(attn_l)
    a = a.at[:, 1].set(attn_r)
    return a


def kernel(features, edge_index, W1, attn_l1, attn_r1, b1, W2, attn_l2, attn_r2, b2):
    src = edge_index[0]
    dst = edge_index[1]
    a1 = _pack_attn(attn_l1, attn_r1)
    a2 = _pack_attn(attn_l2, attn_r2)

    fa, fb, s = _matmul_scores(features[:, :128], features[:, 128:],
                               W1[:128], W1[128:], a1)
    h0, h1 = _edge_phase(fa, fb, s[:, 0], s[:, 1], src, dst, b1, True)

    fa, fb, s = _matmul_scores(h0, h1, W2[:128], W2[128:], a2)
    o0, o1 = _edge_phase(fa, fb, s[:, 0], s[:, 1], src, dst, b2, False)
    return jnp.concatenate([o0, o1], axis=1)


# speedup vs baseline: 10.9831x; 8.7448x over previous
import functools

import jax
import jax.numpy as jnp
from jax import lax
from jax.experimental import pallas as pl
from jax.experimental.pallas import tpu as pltpu
from jax.experimental.pallas import tpu_sc as plsc

N = 10000
NP = 10112
SP = NP // 16
E = 160000
K = 128
HM = 40
NMEGA = 2
NCH = HM * NMEGA
EP = 16 * NCH * K
DIM = 256
HALF = 128
BN1 = 1000
BN2 = 632


def _mm1_body(x_ref, w_ref, a_ref, f_ref, s_ref):
    feat = jnp.dot(x_ref[...], w_ref[...], preferred_element_type=jnp.float32)
    f_ref[0] = feat[:, :HALF]
    f_ref[1] = feat[:, HALF:]
    s_ref[...] = jnp.dot(feat, a_ref[...], preferred_element_type=jnp.float32)


def _mm1(x, w, a):
    return pl.pallas_call(
        _mm1_body,
        grid=(N // BN1,),
        in_specs=[
            pl.BlockSpec((BN1, DIM), lambda i: (i, 0)),
            pl.BlockSpec((DIM, DIM), lambda i: (0, 0)),
            pl.BlockSpec((DIM, 8), lambda i: (0, 0)),
        ],
        out_specs=[
            pl.BlockSpec((2, BN1, HALF), lambda i: (0, i, 0)),
            pl.BlockSpec((BN1, 8), lambda i: (i, 0)),
        ],
        out_shape=[
            jax.ShapeDtypeStruct((2, N, HALF), jnp.float32),
            jax.ShapeDtypeStruct((N, 8), jnp.float32),
        ],
    )(x, w, a)


def _mm2_body(xa_ref, xb_ref, wa_ref, wb_ref, a_ref, f_ref, s_ref):
    feat = jnp.dot(xa_ref[...], wa_ref[...], preferred_element_type=jnp.float32)
    feat += jnp.dot(xb_ref[...], wb_ref[...], preferred_element_type=jnp.float32)
    f_ref[0] = feat[:, :HALF]
    f_ref[1] = feat[:, HALF:]
    s_ref[...] = jnp.dot(feat, a_ref[...], preferred_element_type=jnp.float32)


def _mm2(xa, xb, wa, wb, a):
    return pl.pallas_call(
        _mm2_body,
        grid=(NP // BN2,),
        in_specs=[
            pl.BlockSpec((BN2, HALF), lambda i: (i, 0)),
            pl.BlockSpec((BN2, HALF), lambda i: (i, 0)),
            pl.BlockSpec((HALF, DIM), lambda i: (0, 0)),
            pl.BlockSpec((HALF, DIM), lambda i: (0, 0)),
            pl.BlockSpec((DIM, 8), lambda i: (0, 0)),
        ],
        out_specs=[
            pl.BlockSpec((2, BN2, HALF), lambda i: (0, i, 0)),
            pl.BlockSpec((BN2, 8), lambda i: (i, 0)),
        ],
        out_shape=[
            jax.ShapeDtypeStruct((2, NP, HALF), jnp.float32),
            jax.ShapeDtypeStruct((NP, 8), jnp.float32),
        ],
    )(xa, xb, wa, wb, a)


def _sc_body(elu, tabA, tabB, elh, erh, srcw, dstw, bias2, out,
             src_v, dst_v, r0, r1, els_b, erd_b, ee_b, den_b, bias_v, zv, idx_b,
             el_s, er_s, den_s, acc_s, sem0, sem1):
    cid = lax.axis_index("c")
    sid = lax.axis_index("s")
    zero16 = jnp.zeros((16,), jnp.float32)
    base = sid * SP

    @pl.when(sid == 0)
    def _():
        pltpu.sync_copy(elh, el_s)

    @pl.when(sid == 1)
    def _():
        pltpu.sync_copy(erh, er_s)

    pltpu.sync_copy(bias2.at[cid], bias_v)

    def _z_row(r, carry):
        for j in range(8):
            r0[r, pl.ds(j * 16, 16)] = zero16
        return carry
    lax.fori_loop(0, K, _z_row, 0)
    for i in range(40):
        zv[pl.ds(i * 16, 16)] = zero16
    nfull = SP // K
    rem = SP - nfull * K
    for i in range(nfull):
        pltpu.sync_copy(r0, acc_s.at[pl.ds(base + i * K, K)])
    pltpu.sync_copy(r0.at[pl.ds(0, rem)], acc_s.at[pl.ds(base + nfull * K, rem)])
    pltpu.sync_copy(zv.at[pl.ds(0, SP)], den_s.at[pl.ds(base, SP)])
    plsc.subcore_barrier()

    rows = (r0, r1)
    sems = (sem0, sem1)

    def _start_gather(c, buf, sem):
        @pl.when(cid == 0)
        def _():
            pltpu.async_copy(tabA.at[src_v.at[c]], buf, sem)

        @pl.when(cid == 1)
        def _():
            pltpu.async_copy(tabB.at[src_v.at[c]], buf, sem)

    for m in range(NMEGA):
        pltpu.sync_copy(srcw.at[sid, pl.ds(m * HM, HM)], src_v)
        pltpu.sync_copy(dstw.at[sid, pl.ds(m * HM, HM)], dst_v)
        _start_gather(0, r0, sem0)
        _start_gather(1, r1, sem1)

        def _chunk(g, carry):
            for b in range(2):
                c = g * 2 + b
                pltpu.sync_copy(el_s.at[src_v.at[c]], els_b)
                pltpu.sync_copy(er_s.at[dst_v.at[c]], erd_b)
                for j in range(K // 16):
                    sl = pl.ds(j * 16, 16)
                    e = els_b[sl] + erd_b[sl]
                    e = jnp.where(e > 0, e, 0.2 * e)
                    ee = jnp.exp(e)
                    ee_b[sl] = jnp.where(dst_v[c, sl] >= N, 0.0, ee)
                for j in range(K // 16):
                    sl = pl.ds(j * 16, 16)
                    idx_b[sl] = dst_v[c, sl]
                pltpu.sync_copy(ee_b, den_s.at[idx_b], add=True)

                pltpu.make_async_copy(
                    tabA.at[src_v.at[c]], rows[b], sems[b]).wait()

                def _scale(g2, carry2):
                    a16 = ee_b[pl.ds(g2 * 16, 16)]
                    for i in range(16):
                        r = g2 * 16 + i
                        av = jnp.full((16,), a16[i], jnp.float32)
                        for j in range(8):
                            sl = pl.ds(j * 16, 16)
                            rows[b][r, sl] = rows[b][r, sl] * av
                    return carry2
                lax.fori_loop(0, K // 16, _scale, 0)
                pltpu.sync_copy(rows[b], acc_s.at[idx_b], add=True)
                nc = c + 2

                @pl.when(nc < HM)
                def _():
                    _start_gather(nc, rows[b], sems[b])
            return carry
        lax.fori_loop(0, HM // 2, _chunk, 0)
    plsc.subcore_barrier()

    def _norm(g2, carry):
        d16 = jnp.maximum(den_b[pl.ds(g2 * 16, 16)], 1e-9)
        inv16 = 1.0 / d16
        for i2 in range(16):
            r = g2 * 16 + i2
            dv = jnp.full((16,), inv16[i2], jnp.float32)
            for j in range(8):
                sl = pl.ds(j * 16, 16)
                v = r0[r, sl] * dv + bias_v[sl]
                if elu:
                    v = jnp.where(v > 0, v, jnp.exp(jnp.minimum(v, 0.0)) - 1.0)
                r0[r, sl] = v
        return carry

    def _copyout(cnt, i, carry):
        b0 = base + i * K
        pltpu.sync_copy(acc_s.at[pl.ds(b0, cnt)], r0.at[pl.ds(0, cnt)])
        pltpu.sync_copy(den_s.at[pl.ds(b0, cnt)], den_b.at[pl.ds(0, cnt)])
        lax.fori_loop(0, (cnt + 15) // 16, _norm, 0)
        pltpu.sync_copy(r0.at[pl.ds(0, cnt)], out.at[cid, pl.ds(b0, cnt)])
        return carry

    lax.fori_loop(0, nfull, functools.partial(_copyout, K), 0)
    _copyout(rem, nfull, 0)


def _make_sc(elu):
    mesh = plsc.VectorSubcoreMesh(core_axis_name="c", subcore_axis_name="s")
    return pl.kernel(
        functools.partial(_sc_body, elu),
        out_type=jax.ShapeDtypeStruct((2, NP, HALF), jnp.float32),
        mesh=mesh,
        compiler_params=pltpu.CompilerParams(needs_layout_passes=False),
        scratch_types=[
            pltpu.VMEM((HM, K), jnp.int32),
            pltpu.VMEM((HM, K), jnp.int32),
            pltpu.VMEM((K, HALF), jnp.float32),
            pltpu.VMEM((K, HALF), jnp.float32),
            pltpu.VMEM((K,), jnp.float32),
            pltpu.VMEM((K,), jnp.float32),
            pltpu.VMEM((K,), jnp.float32),
            pltpu.VMEM((K,), jnp.float32),
            pltpu.VMEM((HALF,), jnp.float32),
            pltpu.VMEM((640,), jnp.float32),
            pltpu.VMEM((K,), jnp.int32),
            pltpu.VMEM_SHARED((NP,), jnp.float32),
            pltpu.VMEM_SHARED((NP,), jnp.float32),
            pltpu.VMEM_SHARED((NP,), jnp.float32),
            pltpu.VMEM_SHARED((NP, HALF), jnp.float32),
            pltpu.SemaphoreType.DMA,
            pltpu.SemaphoreType.DMA,
        ],
    )


_sc_layer1 = _make_sc(True)
_sc_layer2 = _make_sc(False)


def _pack_attn(attn_l, attn_r):
    a = jnp.zeros((DIM, 8), jnp.float32)
    a = a.at[:, 0].set(attn_l)
    a = a.at[:, 1].set(attn_r)
    return a


@jax.jit
def kernel(features, edge_index, W1, attn_l1, attn_r1, b1, W2, attn_l2, attn_r2, b2):
    src = edge_index[0]
    dst = edge_index[1]
    pad = EP - E
    srcw = jnp.concatenate([src, jnp.zeros((pad,), jnp.int32)]).reshape(16, NCH, K)
    dstw = jnp.concatenate([dst, jnp.full((pad,), N, jnp.int32)]).reshape(16, NCH, K)

    f2, s = _mm1(features, W1, _pack_attn(attn_l1, attn_r1))
    b1s = jnp.stack([b1[:HALF], b1[HALF:]])
    zpad = jnp.zeros((NP - N,), jnp.float32)
    el1 = jnp.concatenate([s[:, 0], zpad])
    er1 = jnp.concatenate([s[:, 1], zpad])
    h = _sc_layer1(f2[0], f2[1], el1, er1, srcw, dstw, b1s)

    f2b, s2 = _mm2(h[0], h[1], W2[:HALF], W2[HALF:], _pack_attn(attn_l2, attn_r2))
    b2s = jnp.stack([b2[:HALF], b2[HALF:]])
    o = _sc_layer2(f2b[0], f2b[1], s2[:, 0], s2[:, 1], srcw, dstw, b2s)
    return jnp.concatenate([o[0, :N], o[1, :N]], axis=1)
